# baseline (speedup 1.0000x reference)
import functools

import jax
import jax.numpy as jnp
from jax import lax
from jax.experimental import pallas as pl
from jax.experimental.pallas import tpu as pltpu
from jax.experimental.pallas import tpu_sc as plsc

N = 10000
E = 320000
F = 128
H = 256
G = 64
C = 10

NC = 2
NS = 16
CHUNK = 56
R = 4
EDGES_PER_W = E // (NC * NS)
NCHUNK = 184
EPW_PAD = NCHUNK * CHUNK
N_PAD = N + 8
NDUMP = 8
NSRC_SPREAD = 64
RPS = 624
RPS_LAST = N - (NS - 1) * RPS

BLK = 1000
NBLK = N // BLK


def _gather_scatter(h, src3d, dst3d, zrows):
    mesh = plsc.VectorSubcoreMesh(core_axis_name="c", subcore_axis_name="s")

    @functools.partial(
        pl.kernel,
        out_type=jax.ShapeDtypeStruct((NC, N, F), jnp.float32),
        mesh=mesh,
        scratch_types=[
            pltpu.VMEM((EPW_PAD,), jnp.int32),
            pltpu.VMEM((EPW_PAD,), jnp.int32),
            pltpu.VMEM((R, CHUNK, F), jnp.float32),
            pltpu.VMEM_SHARED((N_PAD, F), jnp.float32),
            pltpu.SemaphoreType.DMA,
            pltpu.SemaphoreType.DMA,
        ],
    )
    def k(h_hbm, src_hbm, dst_hbm, z_hbm, out_hbm, sidx, didx, rows, agg,
          gsem, ssem):
        cid = lax.axis_index("c")
        sid = lax.axis_index("s")
        wid = cid * NS + sid
        r0 = sid * RPS

        def seed(nrows):
            @pl.when(cid == 0)
            def _():
                pltpu.async_copy(h_hbm.at[pl.ds(r0, nrows)],
                                 agg.at[pl.ds(r0, nrows)], ssem)

            @pl.when(cid != 0)
            def _():
                pltpu.async_copy(z_hbm.at[pl.ds(0, nrows)],
                                 agg.at[pl.ds(r0, nrows)], ssem)

        def seed_wait(nrows):
            pltpu.make_async_copy(z_hbm.at[pl.ds(0, nrows)],
                                  agg.at[pl.ds(r0, nrows)], ssem).wait()

        s0 = pl.multiple_of(wid * EPW_PAD, 8)
        pltpu.async_copy(src_hbm.at[pl.ds(s0, EPW_PAD)], sidx, gsem)
        pltpu.async_copy(dst_hbm.at[pl.ds(s0, EPW_PAD)], didx, gsem)

        @pl.when(sid < NS - 1)
        def _():
            seed(RPS)
            seed_wait(RPS)

        @pl.when(sid == NS - 1)
        def _():
            seed(RPS_LAST)
            seed_wait(RPS_LAST)

        pltpu.make_async_copy(src_hbm.at[pl.ds(s0, EPW_PAD)], sidx,
                              gsem).wait()
        pltpu.make_async_copy(dst_hbm.at[pl.ds(s0, EPW_PAD)], didx,
                              gsem).wait()
        plsc.subcore_barrier()

        def idxs(ref, j):
            return ref.at[pl.ds(pl.multiple_of(j * CHUNK, 8), CHUNK)]

        def fire_g(b, j):
            pltpu.async_copy(h_hbm.at[idxs(sidx, j)], rows.at[b], gsem)

        def drain_g(b, j):
            pltpu.make_async_copy(h_hbm.at[idxs(sidx, j)], rows.at[b],
                                  gsem).wait()

        def fire_s(b, j):
            pltpu.async_copy(rows.at[b], agg.at[idxs(didx, j)], ssem,
                             add=True)

        def drain_s(b, j):
            pltpu.make_async_copy(rows.at[b], agg.at[idxs(didx, j)],
                                  ssem).wait()

        for b in range(R):
            fire_g(b, b)

        @pl.loop(0, NCHUNK, step=R)
        def _(i):
            for b in range(R):
                drain_g(b, i + b)
                fire_s(b, i + b)
            for b in range(R):
                drain_s(b, i + b)

                @pl.when(i + b + R < NCHUNK)
                def _():
                    fire_g(b, i + b + R)

        plsc.subcore_barrier()

        @pl.when(sid < NS - 1)
        def _():
            pltpu.sync_copy(agg.at[pl.ds(r0, RPS)],
                            out_hbm.at[cid, pl.ds(r0, RPS)])

        @pl.when(sid == NS - 1)
        def _():
            pltpu.sync_copy(agg.at[pl.ds(r0, RPS_LAST)],
                            out_hbm.at[cid, pl.ds(r0, RPS_LAST)])

    return k(h, src3d, dst3d, zrows)


def _mlp(agg, w1, w2):
    def body(a0_ref, a1_ref, w1_ref, w2_ref, o_ref):
        z = a0_ref[0] + a1_ref[0]
        t = jnp.maximum(
            jnp.dot(z, w1_ref[...], preferred_element_type=jnp.float32), 0.0)
        o_ref[...] = jnp.maximum(
            jnp.dot(t, w2_ref[...], preferred_element_type=jnp.float32), 0.0)

    return pl.pallas_call(
        body,
        grid=(NBLK,),
        in_specs=[
            pl.BlockSpec((1, BLK, F), lambda i: (0, i, 0)),
            pl.BlockSpec((1, BLK, F), lambda i: (1, i, 0)),
            pl.BlockSpec((F, H), lambda i: (0, 0)),
            pl.BlockSpec((H, F), lambda i: (0, 0)),
        ],
        out_specs=pl.BlockSpec((BLK, F), lambda i: (i, 0)),
        out_shape=jax.ShapeDtypeStruct((N, F), jnp.float32),
    )(agg, agg, w1, w2)


def _final(agg, batch2d, w1, w2, fcw, fcb2d):
    def body(a0_ref, a1_ref, b_ref, w1_ref, w2_ref, fw_ref, fb_ref, o_ref,
             pool_ref):
        i = pl.program_id(0)

        @pl.when(i == 0)
        def _():
            pool_ref[...] = jnp.zeros_like(pool_ref)

        z = a0_ref[0] + a1_ref[0]
        t = jnp.maximum(
            jnp.dot(z, w1_ref[...], preferred_element_type=jnp.float32), 0.0)
        h4 = jnp.maximum(
            jnp.dot(t, w2_ref[...], preferred_element_type=jnp.float32), 0.0)
        gids = lax.broadcasted_iota(jnp.int32, (BLK, G), 1)
        onehot = (b_ref[...] == gids).astype(jnp.float32)
        pool_ref[...] += lax.dot_general(
            onehot, h4, (((0,), (0,)), ((), ())),
            preferred_element_type=jnp.float32)

        @pl.when(i == NBLK - 1)
        def _():
            logits = jnp.dot(pool_ref[...], fw_ref[...],
                             preferred_element_type=jnp.float32) + fb_ref[...]
            m = jnp.max(logits, axis=1, keepdims=True)
            lse = m + jnp.log(jnp.sum(jnp.exp(logits - m), axis=1,
                                      keepdims=True))
            o_ref[...] = logits - lse

    return pl.pallas_call(
        body,
        grid=(NBLK,),
        in_specs=[
            pl.BlockSpec((1, BLK, F), lambda i: (0, i, 0)),
            pl.BlockSpec((1, BLK, F), lambda i: (1, i, 0)),
            pl.BlockSpec((BLK, 1), lambda i: (i, 0)),
            pl.BlockSpec((F, H), lambda i: (0, 0)),
            pl.BlockSpec((H, F), lambda i: (0, 0)),
            pl.BlockSpec((F, C), lambda i: (0, 0)),
            pl.BlockSpec((1, C), lambda i: (0, 0)),
        ],
        out_specs=pl.BlockSpec((G, C), lambda i: (0, 0)),
        out_shape=jax.ShapeDtypeStruct((G, C), jnp.float32),
        scratch_shapes=[pltpu.VMEM((G, F), jnp.float32)],
    )(agg, agg, batch2d, w1, w2, fcw, fcb2d)


def kernel(x, edge_index, batch, W1_0, W2_0, W1_1, W2_1, W1_2, W2_2, W1_3,
           W2_3, fc_w, fc_b):
    NW = NC * NS
    npad = EPW_PAD - EDGES_PER_W
    srcw = edge_index[0].reshape(NW, EDGES_PER_W)
    dstw = edge_index[1].reshape(NW, EDGES_PER_W)
    spad = jnp.broadcast_to(jnp.arange(npad, dtype=jnp.int32) % NSRC_SPREAD,
                            (NW, npad))
    src3d = jnp.concatenate([srcw, spad], axis=1).reshape(NW * EPW_PAD)
    dpad = jnp.broadcast_to(N + (jnp.arange(npad, dtype=jnp.int32) % NDUMP),
                            (NW, npad))
    dst3d = jnp.concatenate([dstw, dpad], axis=1).reshape(NW * EPW_PAD)
    zrows = jnp.zeros((RPS_LAST, F), jnp.float32)
    batch2d = batch.reshape(N, 1)
    fcb2d = fc_b.reshape(1, C)

    h = x
    for (w1, w2) in [(W1_0, W2_0), (W1_1, W2_1), (W1_2, W2_2)]:
        agg = _gather_scatter(h, src3d, dst3d, zrows)
        h = _mlp(agg, w1, w2)
    agg = _gather_scatter(h, src3d, dst3d, zrows)
    return _final(agg, batch2d, W1_3, W2_3, fc_w, fcb2d)


# speedup vs baseline: 1.0416x; 1.0416x over previous
import functools

import jax
import jax.numpy as jnp
from jax import lax
from jax.experimental import pallas as pl
from jax.experimental.pallas import tpu as pltpu
from jax.experimental.pallas import tpu_sc as plsc

N = 10000
E = 320000
F = 128
H = 256
G = 64
C = 10

NC = 2
NS = 16
CHUNK = 120
R = 2
EDGES_PER_W = E // (NC * NS)
NCHUNK = 84
EPW_PAD = NCHUNK * CHUNK
N_PAD = N + 8
NDUMP = 8
NSRC_SPREAD = 64
RPS = 624
RPS_LAST = N - (NS - 1) * RPS

BLK = 1000
NBLK = N // BLK


def _gather_scatter(h, src3d, dst3d, zrows):
    mesh = plsc.VectorSubcoreMesh(core_axis_name="c", subcore_axis_name="s")

    @functools.partial(
        pl.kernel,
        out_type=jax.ShapeDtypeStruct((NC, N, F), jnp.float32),
        mesh=mesh,
        scratch_types=[
            pltpu.VMEM((EPW_PAD,), jnp.int32),
            pltpu.VMEM((EPW_PAD,), jnp.int32),
            pltpu.VMEM((R, CHUNK, F), jnp.float32),
            pltpu.VMEM_SHARED((N_PAD, F), jnp.float32),
            pltpu.SemaphoreType.DMA,
            pltpu.SemaphoreType.DMA,
        ],
    )
    def k(h_hbm, src_hbm, dst_hbm, z_hbm, out_hbm, sidx, didx, rows, agg,
          gsem, ssem):
        cid = lax.axis_index("c")
        sid = lax.axis_index("s")
        wid = cid * NS + sid
        r0 = sid * RPS

        def seed(nrows):
            @pl.when(cid == 0)
            def _():
                pltpu.async_copy(h_hbm.at[pl.ds(r0, nrows)],
                                 agg.at[pl.ds(r0, nrows)], ssem)

            @pl.when(cid != 0)
            def _():
                pltpu.async_copy(z_hbm.at[pl.ds(0, nrows)],
                                 agg.at[pl.ds(r0, nrows)], ssem)

        def seed_wait(nrows):
            pltpu.make_async_copy(z_hbm.at[pl.ds(0, nrows)],
                                  agg.at[pl.ds(r0, nrows)], ssem).wait()

        s0 = pl.multiple_of(wid * EPW_PAD, 8)
        pltpu.async_copy(src_hbm.at[pl.ds(s0, EPW_PAD)], sidx, gsem)
        pltpu.async_copy(dst_hbm.at[pl.ds(s0, EPW_PAD)], didx, gsem)

        @pl.when(sid < NS - 1)
        def _():
            seed(RPS)
            seed_wait(RPS)

        @pl.when(sid == NS - 1)
        def _():
            seed(RPS_LAST)
            seed_wait(RPS_LAST)

        pltpu.make_async_copy(src_hbm.at[pl.ds(s0, EPW_PAD)], sidx,
                              gsem).wait()
        pltpu.make_async_copy(dst_hbm.at[pl.ds(s0, EPW_PAD)], didx,
                              gsem).wait()
        plsc.subcore_barrier()

        def idxs(ref, j):
            return ref.at[pl.ds(pl.multiple_of(j * CHUNK, 8), CHUNK)]

        def fire_g(b, j):
            pltpu.async_copy(h_hbm.at[idxs(sidx, j)], rows.at[b], gsem)

        def drain_g(b, j):
            pltpu.make_async_copy(h_hbm.at[idxs(sidx, j)], rows.at[b],
                                  gsem).wait()

        def fire_s(b, j):
            pltpu.async_copy(rows.at[b], agg.at[idxs(didx, j)], ssem,
                             add=True)

        def drain_s(b, j):
            pltpu.make_async_copy(rows.at[b], agg.at[idxs(didx, j)],
                                  ssem).wait()

        for b in range(R):
            fire_g(b, b)

        @pl.loop(0, NCHUNK, step=R)
        def _(i):
            for b in range(R):
                drain_g(b, i + b)
                fire_s(b, i + b)
            for b in range(R):
                drain_s(b, i + b)

                @pl.when(i + b + R < NCHUNK)
                def _():
                    fire_g(b, i + b + R)

        plsc.subcore_barrier()

        @pl.when(sid < NS - 1)
        def _():
            pltpu.sync_copy(agg.at[pl.ds(r0, RPS)],
                            out_hbm.at[cid, pl.ds(r0, RPS)])

        @pl.when(sid == NS - 1)
        def _():
            pltpu.sync_copy(agg.at[pl.ds(r0, RPS_LAST)],
                            out_hbm.at[cid, pl.ds(r0, RPS_LAST)])

    return k(h, src3d, dst3d, zrows)


def _mlp(agg, w1, w2):
    def body(a0_ref, a1_ref, w1_ref, w2_ref, o_ref):
        z = a0_ref[0] + a1_ref[0]
        t = jnp.maximum(
            jnp.dot(z, w1_ref[...], preferred_element_type=jnp.float32), 0.0)
        o_ref[...] = jnp.maximum(
            jnp.dot(t, w2_ref[...], preferred_element_type=jnp.float32), 0.0)

    return pl.pallas_call(
        body,
        grid=(NBLK,),
        in_specs=[
            pl.BlockSpec((1, BLK, F), lambda i: (0, i, 0)),
            pl.BlockSpec((1, BLK, F), lambda i: (1, i, 0)),
            pl.BlockSpec((F, H), lambda i: (0, 0)),
            pl.BlockSpec((H, F), lambda i: (0, 0)),
        ],
        out_specs=pl.BlockSpec((BLK, F), lambda i: (i, 0)),
        out_shape=jax.ShapeDtypeStruct((N, F), jnp.float32),
    )(agg, agg, w1, w2)


def _final(agg, batch2d, w1, w2, fcw, fcb2d):
    def body(a0_ref, a1_ref, b_ref, w1_ref, w2_ref, fw_ref, fb_ref, o_ref,
             pool_ref):
        i = pl.program_id(0)

        @pl.when(i == 0)
        def _():
            pool_ref[...] = jnp.zeros_like(pool_ref)

        z = a0_ref[0] + a1_ref[0]
        t = jnp.maximum(
            jnp.dot(z, w1_ref[...], preferred_element_type=jnp.float32), 0.0)
        h4 = jnp.maximum(
            jnp.dot(t, w2_ref[...], preferred_element_type=jnp.float32), 0.0)
        gids = lax.broadcasted_iota(jnp.int32, (BLK, G), 1)
        onehot = (b_ref[...] == gids).astype(jnp.float32)
        pool_ref[...] += lax.dot_general(
            onehot, h4, (((0,), (0,)), ((), ())),
            preferred_element_type=jnp.float32)

        @pl.when(i == NBLK - 1)
        def _():
            logits = jnp.dot(pool_ref[...], fw_ref[...],
                             preferred_element_type=jnp.float32) + fb_ref[...]
            m = jnp.max(logits, axis=1, keepdims=True)
            lse = m + jnp.log(jnp.sum(jnp.exp(logits - m), axis=1,
                                      keepdims=True))
            o_ref[...] = logits - lse

    return pl.pallas_call(
        body,
        grid=(NBLK,),
        in_specs=[
            pl.BlockSpec((1, BLK, F), lambda i: (0, i, 0)),
            pl.BlockSpec((1, BLK, F), lambda i: (1, i, 0)),
            pl.BlockSpec((BLK, 1), lambda i: (i, 0)),
            pl.BlockSpec((F, H), lambda i: (0, 0)),
            pl.BlockSpec((H, F), lambda i: (0, 0)),
            pl.BlockSpec((F, C), lambda i: (0, 0)),
            pl.BlockSpec((1, C), lambda i: (0, 0)),
        ],
        out_specs=pl.BlockSpec((G, C), lambda i: (0, 0)),
        out_shape=jax.ShapeDtypeStruct((G, C), jnp.float32),
        scratch_shapes=[pltpu.VMEM((G, F), jnp.float32)],
    )(agg, agg, batch2d, w1, w2, fcw, fcb2d)


def kernel(x, edge_index, batch, W1_0, W2_0, W1_1, W2_1, W1_2, W2_2, W1_3,
           W2_3, fc_w, fc_b):
    NW = NC * NS
    npad = EPW_PAD - EDGES_PER_W
    srcw = edge_index[0].reshape(NW, EDGES_PER_W)
    dstw = edge_index[1].reshape(NW, EDGES_PER_W)
    spad = jnp.broadcast_to(jnp.arange(npad, dtype=jnp.int32) % NSRC_SPREAD,
                            (NW, npad))
    src3d = jnp.concatenate([srcw, spad], axis=1).reshape(NW * EPW_PAD)
    dpad = jnp.broadcast_to(N + (jnp.arange(npad, dtype=jnp.int32) % NDUMP),
                            (NW, npad))
    dst3d = jnp.concatenate([dstw, dpad], axis=1).reshape(NW * EPW_PAD)
    zrows = jnp.zeros((RPS_LAST, F), jnp.float32)
    batch2d = batch.reshape(N, 1)
    fcb2d = fc_b.reshape(1, C)

    h = x
    for (w1, w2) in [(W1_0, W2_0), (W1_1, W2_1), (W1_2, W2_2)]:
        agg = _gather_scatter(h, src3d, dst3d, zrows)
        h = _mlp(agg, w1, w2)
    agg = _gather_scatter(h, src3d, dst3d, zrows)
    return _final(agg, batch2d, W1_3, W2_3, fc_w, fcb2d)


# speedup vs baseline: 1.0594x; 1.0171x over previous
import functools

import jax
import jax.numpy as jnp
from jax import lax
from jax.experimental import pallas as pl
from jax.experimental.pallas import tpu as pltpu
from jax.experimental.pallas import tpu_sc as plsc

N = 10000
E = 320000
F = 128
H = 256
G = 64
C = 10

NC = 2
NS = 16
CHUNK = 80
R = 3
EDGES_PER_W = E // (NC * NS)
NCHUNK = 126
EPW_PAD = NCHUNK * CHUNK
N_PAD = N + 8
NDUMP = 8
NSRC_SPREAD = 64
RPS = 624
RPS_LAST = N - (NS - 1) * RPS

BLK = 1000
NBLK = N // BLK


def _gather_scatter(h, src3d, dst3d, zrows):
    mesh = plsc.VectorSubcoreMesh(core_axis_name="c", subcore_axis_name="s")

    @functools.partial(
        pl.kernel,
        out_type=jax.ShapeDtypeStruct((NC, N, F), jnp.float32),
        mesh=mesh,
        scratch_types=[
            pltpu.VMEM((EPW_PAD,), jnp.int32),
            pltpu.VMEM((EPW_PAD,), jnp.int32),
            pltpu.VMEM((R, CHUNK, F), jnp.float32),
            pltpu.VMEM_SHARED((N_PAD, F), jnp.float32),
            pltpu.SemaphoreType.DMA,
            pltpu.SemaphoreType.DMA,
        ],
    )
    def k(h_hbm, src_hbm, dst_hbm, z_hbm, out_hbm, sidx, didx, rows, agg,
          gsem, ssem):
        cid = lax.axis_index("c")
        sid = lax.axis_index("s")
        wid = cid * NS + sid
        r0 = sid * RPS

        def seed(nrows):
            @pl.when(cid == 0)
            def _():
                pltpu.async_copy(h_hbm.at[pl.ds(r0, nrows)],
                                 agg.at[pl.ds(r0, nrows)], ssem)

            @pl.when(cid != 0)
            def _():
                pltpu.async_copy(z_hbm.at[pl.ds(0, nrows)],
                                 agg.at[pl.ds(r0, nrows)], ssem)

        def seed_wait(nrows):
            pltpu.make_async_copy(z_hbm.at[pl.ds(0, nrows)],
                                  agg.at[pl.ds(r0, nrows)], ssem).wait()

        s0 = pl.multiple_of(wid * EPW_PAD, 8)
        pltpu.async_copy(src_hbm.at[pl.ds(s0, EPW_PAD)], sidx, gsem)
        pltpu.async_copy(dst_hbm.at[pl.ds(s0, EPW_PAD)], didx, gsem)

        @pl.when(sid < NS - 1)
        def _():
            seed(RPS)
            seed_wait(RPS)

        @pl.when(sid == NS - 1)
        def _():
            seed(RPS_LAST)
            seed_wait(RPS_LAST)

        pltpu.make_async_copy(src_hbm.at[pl.ds(s0, EPW_PAD)], sidx,
                              gsem).wait()
        pltpu.make_async_copy(dst_hbm.at[pl.ds(s0, EPW_PAD)], didx,
                              gsem).wait()
        plsc.subcore_barrier()

        def idxs(ref, j):
            return ref.at[pl.ds(pl.multiple_of(j * CHUNK, 8), CHUNK)]

        def fire_g(b, j):
            pltpu.async_copy(h_hbm.at[idxs(sidx, j)], rows.at[b], gsem)

        def drain_g(b, j):
            pltpu.make_async_copy(h_hbm.at[idxs(sidx, j)], rows.at[b],
                                  gsem).wait()

        def fire_s(b, j):
            pltpu.async_copy(rows.at[b], agg.at[idxs(didx, j)], ssem,
                             add=True)

        def drain_s(b, j):
            pltpu.make_async_copy(rows.at[b], agg.at[idxs(didx, j)],
                                  ssem).wait()

        for b in range(R):
            fire_g(b, b)

        @pl.loop(0, NCHUNK, step=R)
        def _(i):
            for b in range(R):
                drain_g(b, i + b)
                fire_s(b, i + b)
            for b in range(R):
                drain_s(b, i + b)

                @pl.when(i + b + R < NCHUNK)
                def _():
                    fire_g(b, i + b + R)

        plsc.subcore_barrier()

        @pl.when(sid < NS - 1)
        def _():
            pltpu.sync_copy(agg.at[pl.ds(r0, RPS)],
                            out_hbm.at[cid, pl.ds(r0, RPS)])

        @pl.when(sid == NS - 1)
        def _():
            pltpu.sync_copy(agg.at[pl.ds(r0, RPS_LAST)],
                            out_hbm.at[cid, pl.ds(r0, RPS_LAST)])

    return k(h, src3d, dst3d, zrows)


def _mlp(agg, w1, w2):
    def body(a0_ref, a1_ref, w1_ref, w2_ref, o_ref):
        z = a0_ref[0] + a1_ref[0]
        t = jnp.maximum(
            jnp.dot(z, w1_ref[...], preferred_element_type=jnp.float32), 0.0)
        o_ref[...] = jnp.maximum(
            jnp.dot(t, w2_ref[...], preferred_element_type=jnp.float32), 0.0)

    return pl.pallas_call(
        body,
        grid=(NBLK,),
        in_specs=[
            pl.BlockSpec((1, BLK, F), lambda i: (0, i, 0)),
            pl.BlockSpec((1, BLK, F), lambda i: (1, i, 0)),
            pl.BlockSpec((F, H), lambda i: (0, 0)),
            pl.BlockSpec((H, F), lambda i: (0, 0)),
        ],
        out_specs=pl.BlockSpec((BLK, F), lambda i: (i, 0)),
        out_shape=jax.ShapeDtypeStruct((N, F), jnp.float32),
    )(agg, agg, w1, w2)


def _final(agg, batch2d, w1, w2, fcw, fcb2d):
    def body(a0_ref, a1_ref, b_ref, w1_ref, w2_ref, fw_ref, fb_ref, o_ref,
             pool_ref):
        i = pl.program_id(0)

        @pl.when(i == 0)
        def _():
            pool_ref[...] = jnp.zeros_like(pool_ref)

        z = a0_ref[0] + a1_ref[0]
        t = jnp.maximum(
            jnp.dot(z, w1_ref[...], preferred_element_type=jnp.float32), 0.0)
        h4 = jnp.maximum(
            jnp.dot(t, w2_ref[...], preferred_element_type=jnp.float32), 0.0)
        gids = lax.broadcasted_iota(jnp.int32, (BLK, G), 1)
        onehot = (b_ref[...] == gids).astype(jnp.float32)
        pool_ref[...] += lax.dot_general(
            onehot, h4, (((0,), (0,)), ((), ())),
            preferred_element_type=jnp.float32)

        @pl.when(i == NBLK - 1)
        def _():
            logits = jnp.dot(pool_ref[...], fw_ref[...],
                             preferred_element_type=jnp.float32) + fb_ref[...]
            m = jnp.max(logits, axis=1, keepdims=True)
            lse = m + jnp.log(jnp.sum(jnp.exp(logits - m), axis=1,
                                      keepdims=True))
            o_ref[...] = logits - lse

    return pl.pallas_call(
        body,
        grid=(NBLK,),
        in_specs=[
            pl.BlockSpec((1, BLK, F), lambda i: (0, i, 0)),
            pl.BlockSpec((1, BLK, F), lambda i: (1, i, 0)),
            pl.BlockSpec((BLK, 1), lambda i: (i, 0)),
            pl.BlockSpec((F, H), lambda i: (0, 0)),
            pl.BlockSpec((H, F), lambda i: (0, 0)),
            pl.BlockSpec((F, C), lambda i: (0, 0)),
            pl.BlockSpec((1, C), lambda i: (0, 0)),
        ],
        out_specs=pl.BlockSpec((G, C), lambda i: (0, 0)),
        out_shape=jax.ShapeDtypeStruct((G, C), jnp.float32),
        scratch_shapes=[pltpu.VMEM((G, F), jnp.float32)],
    )(agg, agg, batch2d, w1, w2, fcw, fcb2d)


def kernel(x, edge_index, batch, W1_0, W2_0, W1_1, W2_1, W1_2, W2_2, W1_3,
           W2_3, fc_w, fc_b):
    NW = NC * NS
    npad = EPW_PAD - EDGES_PER_W
    srcw = edge_index[0].reshape(NW, EDGES_PER_W)
    dstw = edge_index[1].reshape(NW, EDGES_PER_W)
    spad = jnp.broadcast_to(jnp.arange(npad, dtype=jnp.int32) % NSRC_SPREAD,
                            (NW, npad))
    src3d = jnp.concatenate([srcw, spad], axis=1).reshape(NW * EPW_PAD)
    dpad = jnp.broadcast_to(N + (jnp.arange(npad, dtype=jnp.int32) % NDUMP),
                            (NW, npad))
    dst3d = jnp.concatenate([dstw, dpad], axis=1).reshape(NW * EPW_PAD)
    zrows = jnp.zeros((RPS_LAST, F), jnp.float32)
    batch2d = batch.reshape(N, 1)
    fcb2d = fc_b.reshape(1, C)

    h = x
    for (w1, w2) in [(W1_0, W2_0), (W1_1, W2_1), (W1_2, W2_2)]:
        agg = _gather_scatter(h, src3d, dst3d, zrows)
        h = _mlp(agg, w1, w2)
    agg = _gather_scatter(h, src3d, dst3d, zrows)
    return _final(agg, batch2d, W1_3, W2_3, fc_w, fcb2d)
